# Initial kernel scaffold; baseline (speedup 1.0000x reference)
import functools

import jax
import jax.numpy as jnp
from jax import lax
from jax.experimental import pallas as pl
from jax.experimental.pallas import tpu as pltpu
from jax.experimental.pallas import tpu_sc as plsc

TOPK = 2
E = 8
H = 768
I = 2048
T = 2048

B = 256
NT = (T * TOPK) // B + E
P = NT * B

NW = 32
TPW = T // NW


def _routing_body(logits_ref, pos_ref, w_ref, te_ref, tv_ref):
    l = logits_ref[...]
    eidx = lax.broadcasted_iota(jnp.int32, (T, E), 1)
    NEG = jnp.float32(-3.4e38)

    m1 = jnp.max(l, axis=1, keepdims=True)
    a1 = jnp.min(jnp.where(l == m1, eidx, E), axis=1, keepdims=True)
    l2 = jnp.where(eidx == a1, NEG, l)
    m2 = jnp.max(l2, axis=1, keepdims=True)
    a2 = jnp.min(jnp.where(l2 == m2, eidx, E), axis=1, keepdims=True)

    oh0 = (eidx == a1).astype(jnp.float32)
    oh1 = (eidx == a2).astype(jnp.float32)
    both = oh0 + oh1

    ti = lax.broadcasted_iota(jnp.int32, (T, T), 0)
    tj = lax.broadcasted_iota(jnp.int32, (T, T), 1)
    tri = (tj < ti).astype(jnp.float32)
    csum_excl = jnp.dot(tri, both, preferred_element_type=jnp.float32)

    counts = jnp.sum(both, axis=0, keepdims=True)
    nt_e = jnp.floor((counts + (B - 1)) * (1.0 / B))
    tile_off = jnp.cumsum(nt_e, axis=1) - nt_e
    row_off = tile_off * B
    total_tiles = jnp.sum(nt_e, axis=1, keepdims=True)

    row_off_b = jnp.broadcast_to(row_off, (T, E))
    pos0 = jnp.sum(oh0 * (row_off_b + csum_excl), axis=1, keepdims=True)
    pos1 = jnp.sum(oh1 * (row_off_b + csum_excl + oh0), axis=1, keepdims=True)
    pos_ref[...] = jnp.concatenate(
        [pos0.astype(jnp.int32), pos1.astype(jnp.int32)], axis=1)
    w_ref[...] = jnp.concatenate([m1, m2], axis=1)

    tt = lax.broadcasted_iota(jnp.float32, (NT, E), 0)
    te_grid = lax.broadcasted_iota(jnp.float32, (NT, E), 1)
    total_b = jnp.broadcast_to(total_tiles, (NT, E))
    valid = jnp.sum(jnp.where(tt < total_b, 1.0, 0.0),
                    axis=1, keepdims=True) * (1.0 / E)
    ttc = jnp.minimum(tt, total_b - 1.0)
    off_b = jnp.broadcast_to(tile_off, (NT, E))
    nt_b = jnp.broadcast_to(nt_e, (NT, E))
    in_e = jnp.where((ttc >= off_b) & (ttc < off_b + nt_b), 1.0, 0.0)
    te = jnp.sum(in_e * te_grid, axis=1, keepdims=True)
    te_ref[...] = te.astype(jnp.int32)
    tv_ref[...] = valid.astype(jnp.int32)


def _routing(router_logits):
    return pl.pallas_call(
        _routing_body,
        out_shape=(
            jax.ShapeDtypeStruct((T, TOPK), jnp.int32),
            jax.ShapeDtypeStruct((T, TOPK), jnp.float32),
            jax.ShapeDtypeStruct((NT, 1), jnp.int32),
            jax.ShapeDtypeStruct((NT, 1), jnp.int32),
        ),
    )(router_logits)


_SC_MESH = plsc.VectorSubcoreMesh(core_axis_name="c", subcore_axis_name="s")


@functools.partial(
    pl.kernel,
    mesh=_SC_MESH,
    out_type=jax.ShapeDtypeStruct((P, H), jnp.float32),
    scratch_types=[
        pltpu.VMEM((TOPK, TPW), jnp.int32),
        pltpu.VMEM((TPW, H), jnp.float32),
        pltpu.SemaphoreType.DMA,
        pltpu.SemaphoreType.DMA,
    ],
)
def _scatter_x(x_hbm, pos_hbm, xs_hbm, pos_v, rows_v, sem0, sem1):
    wid = lax.axis_index("s") * 2 + lax.axis_index("c")
    base = wid * TPW
    pltpu.sync_copy(pos_hbm.at[wid], pos_v)
    pltpu.sync_copy(x_hbm.at[pl.ds(base, TPW)], rows_v)
    c0 = pltpu.async_copy(rows_v, xs_hbm.at[pos_v.at[0]], sem0)
    c1 = pltpu.async_copy(rows_v, xs_hbm.at[pos_v.at[1]], sem1)
    c0.wait()
    c1.wait()


def _mm_body(te_ref, tv_ref, x_ref, wg_ref, wu_ref, wd_ref, y_ref):
    t = pl.program_id(0)

    @pl.when(tv_ref[t] == 1)
    def _():
        x = x_ref[...]
        g = jnp.dot(x, wg_ref[0], preferred_element_type=jnp.float32)
        u = jnp.dot(x, wu_ref[0], preferred_element_type=jnp.float32)
        a = g * jax.nn.sigmoid(g) * u
        y_ref[...] = jnp.dot(a, wd_ref[0], preferred_element_type=jnp.float32)


def _grouped_mm(te, tv, xs, W_gate, W_up, W_down):
    grid_spec = pltpu.PrefetchScalarGridSpec(
        num_scalar_prefetch=2,
        grid=(NT,),
        in_specs=[
            pl.BlockSpec((B, H), lambda t, te, tv: (t, 0)),
            pl.BlockSpec((1, H, I), lambda t, te, tv: (te[t], 0, 0)),
            pl.BlockSpec((1, H, I), lambda t, te, tv: (te[t], 0, 0)),
            pl.BlockSpec((1, I, H), lambda t, te, tv: (te[t], 0, 0)),
        ],
        out_specs=pl.BlockSpec((B, H), lambda t, te, tv: (t, 0)),
    )
    return pl.pallas_call(
        _mm_body,
        grid_spec=grid_spec,
        out_shape=jax.ShapeDtypeStruct((P, H), jnp.float32),
    )(te, tv, xs, W_gate, W_up, W_down)


@functools.partial(
    pl.kernel,
    mesh=_SC_MESH,
    out_type=(
        jax.ShapeDtypeStruct((T, H), jnp.float32),
        jax.ShapeDtypeStruct((T, H), jnp.float32),
    ),
    scratch_types=[
        pltpu.VMEM((TOPK, TPW), jnp.int32),
        pltpu.VMEM((TPW, H), jnp.float32),
        pltpu.VMEM((TPW, H), jnp.float32),
        pltpu.SemaphoreType.DMA,
        pltpu.SemaphoreType.DMA,
    ],
)
def _gather_y(y_hbm, pos_hbm, g0_hbm, g1_hbm, pos_v, r0_v, r1_v, sem0, sem1):
    wid = lax.axis_index("s") * 2 + lax.axis_index("c")
    base = wid * TPW
    pltpu.sync_copy(pos_hbm.at[wid], pos_v)
    c0 = pltpu.async_copy(y_hbm.at[pos_v.at[0]], r0_v, sem0)
    c1 = pltpu.async_copy(y_hbm.at[pos_v.at[1]], r1_v, sem1)
    c0.wait()
    c1.wait()
    pltpu.sync_copy(r0_v, g0_hbm.at[pl.ds(base, TPW)])
    pltpu.sync_copy(r1_v, g1_hbm.at[pl.ds(base, TPW)])


def _combine_body(w_ref, g0_ref, g1_ref, o_ref):
    w = w_ref[...]
    o_ref[...] = w[:, 0:1] * g0_ref[...] + w[:, 1:2] * g1_ref[...]


def _combine(w, g0, g1):
    blk = 256
    return pl.pallas_call(
        _combine_body,
        grid=(T // blk,),
        in_specs=[
            pl.BlockSpec((blk, TOPK), lambda i: (i, 0)),
            pl.BlockSpec((blk, H), lambda i: (i, 0)),
            pl.BlockSpec((blk, H), lambda i: (i, 0)),
        ],
        out_specs=pl.BlockSpec((blk, H), lambda i: (i, 0)),
        out_shape=jax.ShapeDtypeStruct((T, H), jnp.float32),
    )(w, g0, g1)


def kernel(hidden_states, router_logits, W_gate, W_up, W_down):
    pos, w, te, tv = _routing(router_logits)
    pos_r = pos.reshape(NW, TPW, TOPK).transpose(0, 2, 1)
    xs = _scatter_x(hidden_states, pos_r)
    y = _grouped_mm(te.reshape(NT), tv.reshape(NT), xs, W_gate, W_up, W_down)
    g0, g1 = _gather_y(y, pos_r)
    return _combine(w, g0, g1)


# speedup vs baseline: 2.3910x; 2.3910x over previous
import functools

import jax
import jax.numpy as jnp
from jax import lax
from jax.experimental import pallas as pl
from jax.experimental.pallas import tpu as pltpu
from jax.experimental.pallas import tpu_sc as plsc

TOPK = 2
E = 8
H = 768
I = 2048
T = 2048

B = 256
NT = (T * TOPK) // B + E
P = NT * B

NW = 32
TPW = T // NW


def _routing_body(logits_ref, pos_ref, w_ref, te_ref, tv_ref):
    l = logits_ref[...]
    eidx = lax.broadcasted_iota(jnp.int32, (T, E), 1)
    NEG = jnp.float32(-3.4e38)

    m1 = jnp.max(l, axis=1, keepdims=True)
    a1 = jnp.min(jnp.where(l == m1, eidx, E), axis=1, keepdims=True)
    l2 = jnp.where(eidx == a1, NEG, l)
    m2 = jnp.max(l2, axis=1, keepdims=True)
    a2 = jnp.min(jnp.where(l2 == m2, eidx, E), axis=1, keepdims=True)

    oh0 = (eidx == a1).astype(jnp.float32)
    oh1 = (eidx == a2).astype(jnp.float32)
    both = oh0 + oh1

    ti = lax.broadcasted_iota(jnp.int32, (T, T), 0)
    tj = lax.broadcasted_iota(jnp.int32, (T, T), 1)
    tri = (tj < ti).astype(jnp.float32)
    csum_excl = jnp.dot(tri, both, preferred_element_type=jnp.float32)

    counts = jnp.sum(both, axis=0, keepdims=True)
    nt_e = jnp.floor((counts + (B - 1)) * (1.0 / B))
    ei = lax.broadcasted_iota(jnp.int32, (E, E), 0)
    ej = lax.broadcasted_iota(jnp.int32, (E, E), 1)
    ustri = (ei < ej).astype(jnp.float32)
    tile_off = jnp.dot(nt_e, ustri, preferred_element_type=jnp.float32)
    row_off = tile_off * B
    total_tiles = jnp.sum(nt_e, axis=1, keepdims=True)

    row_off_b = jnp.broadcast_to(row_off, (T, E))
    pos0 = jnp.sum(oh0 * (row_off_b + csum_excl), axis=1, keepdims=True)
    pos1 = jnp.sum(oh1 * (row_off_b + csum_excl + oh0), axis=1, keepdims=True)
    pos_ref[...] = jnp.concatenate(
        [pos0.astype(jnp.int32), pos1.astype(jnp.int32)], axis=1)
    w_ref[...] = jnp.concatenate([m1, m2], axis=1)

    tt = lax.broadcasted_iota(jnp.int32, (NT, E), 0).astype(jnp.float32)
    te_grid = lax.broadcasted_iota(jnp.int32, (NT, E), 1).astype(jnp.float32)
    total_b = jnp.broadcast_to(total_tiles, (NT, E))
    valid = jnp.sum(jnp.where(tt < total_b, 1.0, 0.0),
                    axis=1, keepdims=True) * (1.0 / E)
    ttc = jnp.minimum(tt, total_b - 1.0)
    off_b = jnp.broadcast_to(tile_off, (NT, E))
    nt_b = jnp.broadcast_to(nt_e, (NT, E))
    in_e = jnp.where((ttc >= off_b) & (ttc < off_b + nt_b), 1.0, 0.0)
    te = jnp.sum(in_e * te_grid, axis=1, keepdims=True)
    te_ref[...] = te.astype(jnp.int32)
    tv_ref[...] = valid.astype(jnp.int32)


def _routing(router_logits):
    return pl.pallas_call(
        _routing_body,
        out_shape=(
            jax.ShapeDtypeStruct((T, TOPK), jnp.int32),
            jax.ShapeDtypeStruct((T, TOPK), jnp.float32),
            jax.ShapeDtypeStruct((NT, 1), jnp.int32),
            jax.ShapeDtypeStruct((NT, 1), jnp.int32),
        ),
    )(router_logits)


@functools.cache
def _sc_mesh():
    return plsc.VectorSubcoreMesh(core_axis_name="c", subcore_axis_name="s")


@functools.cache
def _make_scatter_x():
    @functools.partial(
        pl.kernel,
        mesh=_sc_mesh(),
        out_type=jax.ShapeDtypeStruct((P, H), jnp.float32),
        scratch_types=[
            pltpu.VMEM((TOPK, TPW), jnp.int32),
            pltpu.VMEM((TPW, H), jnp.float32),
            pltpu.SemaphoreType.DMA,
            pltpu.SemaphoreType.DMA,
        ],
    )
    def _scatter_x(x_hbm, pos_hbm, xs_hbm, pos_v, rows_v, sem0, sem1):
        wid = lax.axis_index("s") * 2 + lax.axis_index("c")
        base = wid * TPW
        pltpu.sync_copy(pos_hbm.at[wid], pos_v)
        pltpu.sync_copy(x_hbm.at[pl.ds(base, TPW)], rows_v)
        c0 = pltpu.async_copy(rows_v, xs_hbm.at[pos_v.at[0]], sem0)
        c1 = pltpu.async_copy(rows_v, xs_hbm.at[pos_v.at[1]], sem1)
        c0.wait()
        c1.wait()

    return _scatter_x


def _mm_body(te_ref, tv_ref, x_ref, wg_ref, wu_ref, wd_ref, y_ref):
    t = pl.program_id(0)

    @pl.when(tv_ref[t] == 1)
    def _():
        x = x_ref[...]
        g = jnp.dot(x, wg_ref[0], preferred_element_type=jnp.float32)
        u = jnp.dot(x, wu_ref[0], preferred_element_type=jnp.float32)
        a = g * jax.nn.sigmoid(g) * u
        y_ref[...] = jnp.dot(a, wd_ref[0], preferred_element_type=jnp.float32)


def _grouped_mm(te, tv, xs, W_gate, W_up, W_down):
    grid_spec = pltpu.PrefetchScalarGridSpec(
        num_scalar_prefetch=2,
        grid=(NT,),
        in_specs=[
            pl.BlockSpec((B, H), lambda t, te, tv: (t, 0)),
            pl.BlockSpec((1, H, I), lambda t, te, tv: (te[t], 0, 0)),
            pl.BlockSpec((1, H, I), lambda t, te, tv: (te[t], 0, 0)),
            pl.BlockSpec((1, I, H), lambda t, te, tv: (te[t], 0, 0)),
        ],
        out_specs=pl.BlockSpec((B, H), lambda t, te, tv: (t, 0)),
    )
    return pl.pallas_call(
        _mm_body,
        grid_spec=grid_spec,
        out_shape=jax.ShapeDtypeStruct((P, H), jnp.float32),
    )(te, tv, xs, W_gate, W_up, W_down)


@functools.cache
def _make_gather_y():
    @functools.partial(
        pl.kernel,
        mesh=_sc_mesh(),
        out_type=(
            jax.ShapeDtypeStruct((T, H), jnp.float32),
            jax.ShapeDtypeStruct((T, H), jnp.float32),
        ),
        scratch_types=[
            pltpu.VMEM((TOPK, TPW), jnp.int32),
            pltpu.VMEM((TPW, H), jnp.float32),
            pltpu.VMEM((TPW, H), jnp.float32),
            pltpu.SemaphoreType.DMA,
            pltpu.SemaphoreType.DMA,
        ],
    )
    def _gather_y(y_hbm, pos_hbm, g0_hbm, g1_hbm, pos_v, r0_v, r1_v, sem0, sem1):
        wid = lax.axis_index("s") * 2 + lax.axis_index("c")
        base = wid * TPW
        pltpu.sync_copy(pos_hbm.at[wid], pos_v)
        c0 = pltpu.async_copy(y_hbm.at[pos_v.at[0]], r0_v, sem0)
        c1 = pltpu.async_copy(y_hbm.at[pos_v.at[1]], r1_v, sem1)
        c0.wait()
        c1.wait()
        pltpu.sync_copy(r0_v, g0_hbm.at[pl.ds(base, TPW)])
        pltpu.sync_copy(r1_v, g1_hbm.at[pl.ds(base, TPW)])

    return _gather_y


def _combine_body(w_ref, g0_ref, g1_ref, o_ref):
    w = w_ref[...]
    o_ref[...] = w[:, 0:1] * g0_ref[...] + w[:, 1:2] * g1_ref[...]


def _combine(w, g0, g1):
    blk = 256
    return pl.pallas_call(
        _combine_body,
        grid=(T // blk,),
        in_specs=[
            pl.BlockSpec((blk, TOPK), lambda i: (i, 0)),
            pl.BlockSpec((blk, H), lambda i: (i, 0)),
            pl.BlockSpec((blk, H), lambda i: (i, 0)),
        ],
        out_specs=pl.BlockSpec((blk, H), lambda i: (i, 0)),
        out_shape=jax.ShapeDtypeStruct((T, H), jnp.float32),
    )(w, g0, g1)


def kernel(hidden_states, router_logits, W_gate, W_up, W_down):
    pos, w, te, tv = _routing(router_logits)
    pos_r = pos.reshape(NW, TPW, TOPK).transpose(0, 2, 1)
    xs = _make_scatter_x()(hidden_states, pos_r)
    y = _grouped_mm(te.reshape(NT), tv.reshape(NT), xs, W_gate, W_up, W_down)
    g0, g1 = _make_gather_y()(y, pos_r)
    return _combine(w, g0, g1)
